# baseline (speedup 1.0000x reference)
import jax
import jax.numpy as jnp
from jax import lax
from jax.experimental import pallas as pl
from jax.experimental.pallas import tpu as pltpu
from jax.experimental.pallas import tpu_sc as plsc

N = 4096
D = 1024
DF = 4096
E = 8
NA = 2 * N
BM = 256
PMAX = NA + E * BM
NB = PMAX // BM
NBE = 48
TB = 512
BDF = 2048
NEG = -1e30


def _router_body(x_ref, wr_ref, bias_ref, eid_ref, rank_ref, scl_ref,
                 cnt_ref, carry_ref):
    b = pl.program_id(0)

    @pl.when(b == 0)
    def _():
        carry_ref[...] = jnp.zeros_like(carry_ref)

    xb = x_ref[...]
    logits = jax.lax.dot_general(
        xb, wr_ref[...], (((1,), (1,)), ((), ())),
        preferred_element_type=jnp.float32)
    logits = logits + bias_ref[...]
    iota_e = lax.broadcasted_iota(jnp.int32, (TB, E), 1)
    m1 = jnp.max(logits, axis=1, keepdims=True)
    a1 = jnp.min(jnp.where(logits == m1, iota_e, E), axis=1, keepdims=True)
    masked = jnp.where(iota_e == a1, NEG, logits)
    m2 = jnp.max(masked, axis=1, keepdims=True)
    a2 = jnp.min(jnp.where(masked == m2, iota_e, E), axis=1, keepdims=True)
    s0 = 1.0 / (1.0 + jnp.exp(m2 - m1))
    s1 = 1.0 - s0

    oh1 = (iota_e == a1).astype(jnp.float32)
    oh2 = (iota_e == a2).astype(jnp.float32)
    oh = oh1 + oh2
    r_i = lax.broadcasted_iota(jnp.int32, (TB, TB), 0)
    c_i = lax.broadcasted_iota(jnp.int32, (TB, TB), 1)
    stril = (r_i > c_i).astype(jnp.float32)
    carry = carry_ref[0:1, 0:E]
    rank_base = jax.lax.dot_general(
        stril, oh, (((1,), (0,)), ((), ())),
        preferred_element_type=jnp.float32) + carry
    rank0 = jnp.sum(rank_base * oh1, axis=1, keepdims=True)
    rank1 = jnp.sum(rank_base * oh2, axis=1, keepdims=True)

    eid_ref[...] = jnp.concatenate([a1, a2], axis=1)
    rank_ref[...] = jnp.concatenate([rank0, rank1], axis=1).astype(jnp.int32)
    scl_ref[...] = jnp.concatenate([s0, s1], axis=1)
    new_c = carry + jnp.sum(oh, axis=0, keepdims=True)
    carry_ref[0:1, 0:E] = new_c
    cnt = jnp.concatenate(
        [new_c, jnp.zeros((1, 16 - E), jnp.float32)], axis=1)
    cnt_ref[...] = cnt.astype(jnp.int32)


def _router(x_flat, W_router, bias2):
    return pl.pallas_call(
        _router_body,
        grid=(N // TB,),
        in_specs=[
            pl.BlockSpec((TB, D), lambda b: (b, 0)),
            pl.BlockSpec((E, D), lambda b: (0, 0)),
            pl.BlockSpec((1, E), lambda b: (0, 0)),
        ],
        out_specs=[
            pl.BlockSpec((TB, 2), lambda b: (b, 0)),
            pl.BlockSpec((TB, 2), lambda b: (b, 0)),
            pl.BlockSpec((TB, 2), lambda b: (b, 0)),
            pl.BlockSpec((1, 16), lambda b: (0, 0)),
        ],
        out_shape=[
            jax.ShapeDtypeStruct((N, 2), jnp.int32),
            jax.ShapeDtypeStruct((N, 2), jnp.int32),
            jax.ShapeDtypeStruct((N, 2), jnp.float32),
            jax.ShapeDtypeStruct((1, 16), jnp.int32),
        ],
        scratch_shapes=[pltpu.VMEM((8, 128), jnp.float32)],
        compiler_params=pltpu.CompilerParams(
            dimension_semantics=("arbitrary",)),
    )(x_flat, W_router, bias2)


_AS_PER_TILE = NA // 16
_SL_PER_TILE = PMAX // 16


def _sort_body(eid_hbm, rank_hbm, scl_hbm, cnt_hbm,
               st_hbm, ss_hbm, inv_hbm, beh_hbm,
               cnt_v, padoff_v, blkoff_v, eid_v, rank_v, scl_v,
               pos_v, tok_v, sc4_v, tmp_v, tmpf_v, be_v, sh_t, sh_s):
    cid = lax.axis_index("c")
    sid = lax.axis_index("s")

    @pl.when(cid == 0)
    def _():
        base = sid * _AS_PER_TILE
        pltpu.sync_copy(cnt_hbm, cnt_v)
        c = cnt_v[...]
        pc = lax.shift_left(lax.shift_right_logical(c + (BM - 1), 8), 8)
        incl = jnp.cumsum(pc)
        padoff = incl - pc
        padoff_v[...] = padoff
        blkoff_v[...] = lax.shift_right_logical(padoff, 8)
        pltpu.sync_copy(eid_hbm.at[pl.ds(base, _AS_PER_TILE)], eid_v)
        pltpu.sync_copy(rank_hbm.at[pl.ds(base, _AS_PER_TILE)], rank_v)
        pltpu.sync_copy(scl_hbm.at[pl.ds(base, _AS_PER_TILE)], scl_v)

        iota16 = lax.iota(jnp.int32, 16)
        for m in range(_AS_PER_TILE // 16):
            e = eid_v[pl.ds(16 * m, 16)]
            r = rank_v[pl.ds(16 * m, 16)]
            off = plsc.load_gather(padoff_v, [e])
            pos = off + r
            tok = lax.shift_right_logical(base + 16 * m + iota16, 1)
            row, col = m // 8, (m % 8) * 16
            pos_v[row, pl.ds(col, 16)] = pos
            tok_v[row, pl.ds(col, 16)] = tok
            sc4_v[row, pl.ds(col, 16)] = scl_v[pl.ds(16 * m, 16)]

        pltpu.sync_copy(pos_v, inv_hbm.at[sid])

        z16 = jnp.zeros((16,), jnp.int32)
        zf16 = jnp.zeros((16,), jnp.float32)
        for m in range(_SL_PER_TILE // 16):
            tmp_v[pl.ds(16 * m, 16)] = z16
            tmpf_v[pl.ds(16 * m, 16)] = zf16
        sl = pl.ds(sid * _SL_PER_TILE, _SL_PER_TILE)
        pltpu.sync_copy(tmp_v, sh_t.at[sl])
        pltpu.sync_copy(tmpf_v, sh_s.at[sl])
        plsc.subcore_barrier()

        for j in range(4):
            pltpu.sync_copy(tok_v.at[j], sh_t.at[pos_v.at[j]], add=True)
            pltpu.sync_copy(sc4_v.at[j], sh_s.at[pos_v.at[j]], add=True)
        plsc.subcore_barrier()

        pltpu.sync_copy(sh_t.at[sl], tmp_v)
        pltpu.sync_copy(tmp_v, st_hbm.at[sl])
        pltpu.sync_copy(sh_s.at[sl], tmpf_v)
        pltpu.sync_copy(tmpf_v, ss_hbm.at[sl])

        @pl.when(sid == 0)
        def _():
            for m in range(NBE // 16):
                bv = lax.iota(jnp.int32, 16) + 16 * m
                acc = jnp.zeros((16,), jnp.int32)
                for e in range(E):
                    sp = plsc.load_gather(
                        blkoff_v, [jnp.full((16,), e, jnp.int32)])
                    acc = acc + jnp.where(bv >= sp, 1, 0).astype(jnp.int32)
                be_v[pl.ds(16 * m, 16)] = acc - 1
            pltpu.sync_copy(be_v, beh_hbm)


def _sort(eid_f, rank_f, scl_f, cnt16):
    return pl.kernel(
        _sort_body,
        out_type=[
            jax.ShapeDtypeStruct((PMAX,), jnp.int32),
            jax.ShapeDtypeStruct((PMAX,), jnp.float32),
            jax.ShapeDtypeStruct((16, 4, 128), jnp.int32),
            jax.ShapeDtypeStruct((NBE,), jnp.int32),
        ],
        mesh=plsc.VectorSubcoreMesh(core_axis_name="c", subcore_axis_name="s"),
        scratch_types=[
            pltpu.VMEM((16,), jnp.int32),
            pltpu.VMEM((16,), jnp.int32),
            pltpu.VMEM((16,), jnp.int32),
            pltpu.VMEM((_AS_PER_TILE,), jnp.int32),
            pltpu.VMEM((_AS_PER_TILE,), jnp.int32),
            pltpu.VMEM((_AS_PER_TILE,), jnp.float32),
            pltpu.VMEM((4, 128), jnp.int32),
            pltpu.VMEM((4, 128), jnp.int32),
            pltpu.VMEM((4, 128), jnp.float32),
            pltpu.VMEM((_SL_PER_TILE,), jnp.int32),
            pltpu.VMEM((_SL_PER_TILE,), jnp.float32),
            pltpu.VMEM((NBE,), jnp.int32),
            pltpu.VMEM_SHARED((PMAX,), jnp.int32),
            pltpu.VMEM_SHARED((PMAX,), jnp.float32),
        ],
        compiler_params=pltpu.CompilerParams(needs_layout_passes=False),
    )(eid_f, rank_f, scl_f, cnt16)


_G_ROWS = PMAX // 32
_G_CHUNK = 40
_G_NCH = _G_ROWS // _G_CHUNK


def _gather_body(st_hbm, x_hbm, xg_hbm, idxall_v, rows0, rows1,
                 g0, g1, w0, w1):
    wid = lax.axis_index("s") * 2 + lax.axis_index("c")
    rbase = wid * _G_ROWS
    rows = (rows0, rows1)
    gsem = (g0, g1)
    wsem = (w0, w1)
    pltpu.sync_copy(st_hbm.at[pl.ds(rbase, _G_ROWS)], idxall_v)

    def start_gather(c):
        return pltpu.async_copy(
            x_hbm.at[idxall_v.at[pl.ds(_G_CHUNK * c, _G_CHUNK)]],
            rows[c % 2], gsem[c % 2])

    gd = [None] * _G_NCH
    wd = [None] * _G_NCH
    gd[0] = start_gather(0)
    for c in range(_G_NCH):
        if c + 1 < _G_NCH:
            if c + 1 >= 2:
                wd[c - 1].wait()
            gd[c + 1] = start_gather(c + 1)
        gd[c].wait()
        wd[c] = pltpu.async_copy(
            rows[c % 2],
            xg_hbm.at[pl.ds(rbase + _G_CHUNK * c, _G_CHUNK)], wsem[c % 2])
    wd[_G_NCH - 2].wait()
    wd[_G_NCH - 1].wait()


def _gather(sorted_tok, x_i32):
    return pl.kernel(
        _gather_body,
        out_type=jax.ShapeDtypeStruct((PMAX, D // 2), jnp.int32),
        mesh=plsc.VectorSubcoreMesh(core_axis_name="c", subcore_axis_name="s"),
        scratch_types=[
            pltpu.VMEM((_G_ROWS,), jnp.int32),
            pltpu.VMEM((_G_CHUNK, D // 2), jnp.int32),
            pltpu.VMEM((_G_CHUNK, D // 2), jnp.int32),
            pltpu.SemaphoreType.DMA,
            pltpu.SemaphoreType.DMA,
            pltpu.SemaphoreType.DMA,
            pltpu.SemaphoreType.DMA,
        ],
    )(sorted_tok, x_i32)


def _ffn1_body(be_ref, xg_ref, w1_ref, w3_ref, h_ref):
    xb = xg_ref[...].astype(jnp.float32)
    g = jnp.dot(xb, w1_ref[0], preferred_element_type=jnp.float32)
    u = jnp.dot(xb, w3_ref[0], preferred_element_type=jnp.float32)
    h_ref[...] = (g * jax.lax.logistic(g)) * u


def _ffn1(beh, xg, w1, w3):
    return pl.pallas_call(
        _ffn1_body,
        grid_spec=pltpu.PrefetchScalarGridSpec(
            num_scalar_prefetch=1,
            grid=(DF // BDF, NB),
            in_specs=[
                pl.BlockSpec((BM, D), lambda j, b, be: (b, 0)),
                pl.BlockSpec((1, D, BDF), lambda j, b, be: (be[b], 0, j)),
                pl.BlockSpec((1, D, BDF), lambda j, b, be: (be[b], 0, j)),
            ],
            out_specs=pl.BlockSpec((BM, BDF), lambda j, b, be: (b, j)),
        ),
        out_shape=jax.ShapeDtypeStruct((PMAX, DF), jnp.float32),
        compiler_params=pltpu.CompilerParams(
            dimension_semantics=("arbitrary", "arbitrary")),
    )(beh, xg, w1, w3)


def _ffn2_body(be_ref, h_ref, w2_ref, s_ref, ys_ref):
    out = jnp.dot(h_ref[...], w2_ref[0], preferred_element_type=jnp.float32)
    ys_ref[...] = out * s_ref[...]


def _ffn2(beh, h, w2, ssc):
    return pl.pallas_call(
        _ffn2_body,
        grid_spec=pltpu.PrefetchScalarGridSpec(
            num_scalar_prefetch=1,
            grid=(NB,),
            in_specs=[
                pl.BlockSpec((BM, DF), lambda b, be: (b, 0)),
                pl.BlockSpec((1, DF, D), lambda b, be: (be[b], 0, 0)),
                pl.BlockSpec((BM, 1), lambda b, be: (b, 0)),
            ],
            out_specs=pl.BlockSpec((BM, D), lambda b, be: (b, 0)),
        ),
        out_shape=jax.ShapeDtypeStruct((PMAX, D), jnp.float32),
        compiler_params=pltpu.CompilerParams(
            dimension_semantics=("arbitrary",)),
    )(beh, h, w2, ssc)


_C_TOK = N // 32
_C_CHUNK = 16
_C_NCH = _C_TOK // _C_CHUNK


def _combine_body(inv_hbm, ys_hbm, out_hbm, idxall_v, rows0, rows1,
                  out0, out1, g0, g1, w0, w1):
    wid = lax.axis_index("s") * 2 + lax.axis_index("c")
    tokbase = wid * _C_TOK
    rows = (rows0, rows1)
    outs = (out0, out1)
    gsem = (g0, g1)
    wsem = (w0, w1)
    pltpu.sync_copy(inv_hbm.at[pl.ds(2 * tokbase, 2 * _C_TOK)], idxall_v)

    def start_gather(c):
        return pltpu.async_copy(
            ys_hbm.at[idxall_v.at[pl.ds(2 * _C_CHUNK * c, 2 * _C_CHUNK)]],
            rows[c % 2], gsem[c % 2])

    gd = [None] * _C_NCH
    wd = [None] * _C_NCH
    gd[0] = start_gather(0)
    for c in range(_C_NCH):
        if c + 1 < _C_NCH:
            gd[c + 1] = start_gather(c + 1)
        gd[c].wait()
        if c >= 2:
            wd[c - 2].wait()
        rv, ov = rows[c % 2], outs[c % 2]
        for t in range(_C_CHUNK):
            def qbody(q, _, t=t):
                off = q * 16
                a = rv[2 * t, pl.ds(off, 16)]
                bb = rv[2 * t + 1, pl.ds(off, 16)]
                ov[t, pl.ds(off, 16)] = a + bb
                return 0
            lax.fori_loop(0, D // 16, qbody, 0)
        wd[c] = pltpu.async_copy(
            ov, out_hbm.at[pl.ds(tokbase + _C_CHUNK * c, _C_CHUNK)],
            wsem[c % 2])
    wd[_C_NCH - 2].wait()
    wd[_C_NCH - 1].wait()


def _combine(inv_f, ys):
    return pl.kernel(
        _combine_body,
        out_type=jax.ShapeDtypeStruct((N, D), jnp.float32),
        mesh=plsc.VectorSubcoreMesh(core_axis_name="c", subcore_axis_name="s"),
        scratch_types=[
            pltpu.VMEM((2 * _C_TOK,), jnp.int32),
            pltpu.VMEM((2 * _C_CHUNK, D), jnp.float32),
            pltpu.VMEM((2 * _C_CHUNK, D), jnp.float32),
            pltpu.VMEM((_C_CHUNK, D), jnp.float32),
            pltpu.VMEM((_C_CHUNK, D), jnp.float32),
            pltpu.SemaphoreType.DMA,
            pltpu.SemaphoreType.DMA,
            pltpu.SemaphoreType.DMA,
            pltpu.SemaphoreType.DMA,
        ],
    )(inv_f, ys)


def kernel(x, W_router, w1, w3, w2, expert_bias):
    Bs, Ss, Ds = x.shape
    x_flat = x.reshape(N, D)
    bias2 = expert_bias.reshape(1, E)

    eid, rank, scl, cnt = _router(x_flat, W_router, bias2)
    sorted_tok, sorted_scl, inv3, beh = _sort(
        eid.reshape(-1), rank.reshape(-1), scl.reshape(-1), cnt.reshape(16))
    x_i32 = jax.lax.bitcast_convert_type(
        x_flat.astype(jnp.bfloat16).reshape(N, D // 2, 2), jnp.int32)
    xg_i = _gather(sorted_tok, x_i32)
    xg = jax.lax.bitcast_convert_type(xg_i, jnp.bfloat16).reshape(PMAX, D)
    h = _ffn1(beh, xg, w1, w3)
    ys = _ffn2(beh, h, w2, sorted_scl.reshape(PMAX, 1))
    out_flat = _combine(inv3.reshape(-1), ys)

    out = out_flat.reshape(Bs, Ss, Ds)
    aux_loss = jnp.zeros((), dtype=x.dtype)
    return (out, aux_loss)


# speedup vs baseline: 1.3972x; 1.3972x over previous
import jax
import jax.numpy as jnp
from jax import lax
from jax.experimental import pallas as pl
from jax.experimental.pallas import tpu as pltpu
from jax.experimental.pallas import tpu_sc as plsc

N = 4096
D = 1024
DF = 4096
E = 8
NA = 2 * N
BM = 256
PMAX = NA + E * BM
NB = PMAX // BM
NBE = 48
TB = 512
BDF = 2048
NEG = -1e30


def _router_body(x_ref, wr_ref, bias_ref, eid_ref, rank_ref, scl_ref,
                 cnt_ref, carry_ref):
    b = pl.program_id(0)

    @pl.when(b == 0)
    def _():
        carry_ref[...] = jnp.zeros_like(carry_ref)

    xb = x_ref[...]
    logits = jax.lax.dot_general(
        xb, wr_ref[...], (((1,), (1,)), ((), ())),
        preferred_element_type=jnp.float32)
    logits = logits + bias_ref[...]
    iota_e = lax.broadcasted_iota(jnp.int32, (TB, E), 1)
    m1 = jnp.max(logits, axis=1, keepdims=True)
    a1 = jnp.min(jnp.where(logits == m1, iota_e, E), axis=1, keepdims=True)
    masked = jnp.where(iota_e == a1, NEG, logits)
    m2 = jnp.max(masked, axis=1, keepdims=True)
    a2 = jnp.min(jnp.where(masked == m2, iota_e, E), axis=1, keepdims=True)
    s0 = 1.0 / (1.0 + jnp.exp(m2 - m1))
    s1 = 1.0 - s0

    oh1 = (iota_e == a1).astype(jnp.float32)
    oh2 = (iota_e == a2).astype(jnp.float32)
    oh = oh1 + oh2
    r_i = lax.broadcasted_iota(jnp.int32, (TB, TB), 0)
    c_i = lax.broadcasted_iota(jnp.int32, (TB, TB), 1)
    stril = (r_i > c_i).astype(jnp.float32)
    carry = carry_ref[0:1, 0:E]
    rank_base = jax.lax.dot_general(
        stril, oh, (((1,), (0,)), ((), ())),
        preferred_element_type=jnp.float32) + carry
    rank0 = jnp.sum(rank_base * oh1, axis=1, keepdims=True)
    rank1 = jnp.sum(rank_base * oh2, axis=1, keepdims=True)

    eid_ref[...] = jnp.concatenate([a1, a2], axis=1)
    rank_ref[...] = jnp.concatenate([rank0, rank1], axis=1).astype(jnp.int32)
    scl_ref[...] = jnp.concatenate([s0, s1], axis=1)
    new_c = carry + jnp.sum(oh, axis=0, keepdims=True)
    carry_ref[0:1, 0:E] = new_c
    cnt = jnp.concatenate(
        [new_c, jnp.zeros((1, 16 - E), jnp.float32)], axis=1)
    cnt_ref[...] = cnt.astype(jnp.int32)


def _router(x_flat, W_router, bias2):
    return pl.pallas_call(
        _router_body,
        grid=(N // TB,),
        in_specs=[
            pl.BlockSpec((TB, D), lambda b: (b, 0)),
            pl.BlockSpec((E, D), lambda b: (0, 0)),
            pl.BlockSpec((1, E), lambda b: (0, 0)),
        ],
        out_specs=[
            pl.BlockSpec((TB, 2), lambda b: (b, 0)),
            pl.BlockSpec((TB, 2), lambda b: (b, 0)),
            pl.BlockSpec((TB, 2), lambda b: (b, 0)),
            pl.BlockSpec((1, 16), lambda b: (0, 0)),
        ],
        out_shape=[
            jax.ShapeDtypeStruct((N, 2), jnp.int32),
            jax.ShapeDtypeStruct((N, 2), jnp.int32),
            jax.ShapeDtypeStruct((N, 2), jnp.float32),
            jax.ShapeDtypeStruct((1, 16), jnp.int32),
        ],
        scratch_shapes=[pltpu.VMEM((8, 128), jnp.float32)],
        compiler_params=pltpu.CompilerParams(
            dimension_semantics=("arbitrary",)),
    )(x_flat, W_router, bias2)


_AS_PER_TILE = NA // 16
_SL_PER_TILE = PMAX // 16


def _sort_body(eid_hbm, rank_hbm, scl_hbm, cnt_hbm,
               st_hbm, ss_hbm, inv_hbm, beh_hbm,
               cnt_v, padoff_v, blkoff_v, eid_v, rank_v, scl_v,
               pos_v, tok_v, sc4_v, tmp_v, tmpf_v, be_v, sh_t, sh_s):
    cid = lax.axis_index("c")
    sid = lax.axis_index("s")

    @pl.when(cid == 0)
    def _():
        base = sid * _AS_PER_TILE
        pltpu.sync_copy(cnt_hbm, cnt_v)
        c = cnt_v[...]
        pc = lax.shift_left(lax.shift_right_logical(c + (BM - 1), 8), 8)
        incl = jnp.cumsum(pc)
        padoff = incl - pc
        padoff_v[...] = padoff
        blkoff_v[...] = lax.shift_right_logical(padoff, 8)
        pltpu.sync_copy(eid_hbm.at[pl.ds(base, _AS_PER_TILE)], eid_v)
        pltpu.sync_copy(rank_hbm.at[pl.ds(base, _AS_PER_TILE)], rank_v)
        pltpu.sync_copy(scl_hbm.at[pl.ds(base, _AS_PER_TILE)], scl_v)

        iota16 = lax.iota(jnp.int32, 16)
        for m in range(_AS_PER_TILE // 16):
            e = eid_v[pl.ds(16 * m, 16)]
            r = rank_v[pl.ds(16 * m, 16)]
            off = plsc.load_gather(padoff_v, [e])
            pos = off + r
            tok = lax.shift_right_logical(base + 16 * m + iota16, 1)
            row, col = m // 8, (m % 8) * 16
            pos_v[row, pl.ds(col, 16)] = pos
            tok_v[row, pl.ds(col, 16)] = tok
            sc4_v[row, pl.ds(col, 16)] = scl_v[pl.ds(16 * m, 16)]

        pltpu.sync_copy(pos_v, inv_hbm.at[sid])

        z16 = jnp.zeros((16,), jnp.int32)
        zf16 = jnp.zeros((16,), jnp.float32)
        for m in range(_SL_PER_TILE // 16):
            tmp_v[pl.ds(16 * m, 16)] = z16
            tmpf_v[pl.ds(16 * m, 16)] = zf16
        sl = pl.ds(sid * _SL_PER_TILE, _SL_PER_TILE)
        pltpu.sync_copy(tmp_v, sh_t.at[sl])
        pltpu.sync_copy(tmpf_v, sh_s.at[sl])
        plsc.subcore_barrier()

        for j in range(4):
            pltpu.sync_copy(tok_v.at[j], sh_t.at[pos_v.at[j]], add=True)
            pltpu.sync_copy(sc4_v.at[j], sh_s.at[pos_v.at[j]], add=True)
        plsc.subcore_barrier()

        pltpu.sync_copy(sh_t.at[sl], tmp_v)
        pltpu.sync_copy(tmp_v, st_hbm.at[sl])
        pltpu.sync_copy(sh_s.at[sl], tmpf_v)
        pltpu.sync_copy(tmpf_v, ss_hbm.at[sl])

        @pl.when(sid == 0)
        def _():
            for m in range(NBE // 16):
                bv = lax.iota(jnp.int32, 16) + 16 * m
                acc = jnp.zeros((16,), jnp.int32)
                for e in range(E):
                    sp = plsc.load_gather(
                        blkoff_v, [jnp.full((16,), e, jnp.int32)])
                    acc = acc + jnp.where(bv >= sp, 1, 0).astype(jnp.int32)
                be_v[pl.ds(16 * m, 16)] = acc - 1
            pltpu.sync_copy(be_v, beh_hbm)


def _sort(eid_f, rank_f, scl_f, cnt16):
    return pl.kernel(
        _sort_body,
        out_type=[
            jax.ShapeDtypeStruct((PMAX,), jnp.int32),
            jax.ShapeDtypeStruct((PMAX,), jnp.float32),
            jax.ShapeDtypeStruct((16, 4, 128), jnp.int32),
            jax.ShapeDtypeStruct((NBE,), jnp.int32),
        ],
        mesh=plsc.VectorSubcoreMesh(core_axis_name="c", subcore_axis_name="s"),
        scratch_types=[
            pltpu.VMEM((16,), jnp.int32),
            pltpu.VMEM((16,), jnp.int32),
            pltpu.VMEM((16,), jnp.int32),
            pltpu.VMEM((_AS_PER_TILE,), jnp.int32),
            pltpu.VMEM((_AS_PER_TILE,), jnp.int32),
            pltpu.VMEM((_AS_PER_TILE,), jnp.float32),
            pltpu.VMEM((4, 128), jnp.int32),
            pltpu.VMEM((4, 128), jnp.int32),
            pltpu.VMEM((4, 128), jnp.float32),
            pltpu.VMEM((_SL_PER_TILE,), jnp.int32),
            pltpu.VMEM((_SL_PER_TILE,), jnp.float32),
            pltpu.VMEM((NBE,), jnp.int32),
            pltpu.VMEM_SHARED((PMAX,), jnp.int32),
            pltpu.VMEM_SHARED((PMAX,), jnp.float32),
        ],
        compiler_params=pltpu.CompilerParams(needs_layout_passes=False),
    )(eid_f, rank_f, scl_f, cnt16)


PH = PMAX // 2
NBH = NB // 2
_G_ROWS = PH // 32
_G_CHUNK = 40
_G_NCH = _G_ROWS // _G_CHUNK


def _make_gather_body(nhalf):
    def _gather_body(st_hbm, x_hbm, xg_hbm, idxall_v, rows0, rows1,
                     g0, g1, w0, w1):
        wid = lax.axis_index("s") * 2 + lax.axis_index("c")
        sbase = nhalf * PH + wid * _G_ROWS
        rbase = wid * _G_ROWS
        rows = (rows0, rows1)
        gsem = (g0, g1)
        wsem = (w0, w1)
        pltpu.sync_copy(st_hbm.at[pl.ds(sbase, _G_ROWS)], idxall_v)

        def start_gather(c):
            return pltpu.async_copy(
                x_hbm.at[idxall_v.at[pl.ds(_G_CHUNK * c, _G_CHUNK)]],
                rows[c % 2], gsem[c % 2])

        gd = [None] * _G_NCH
        wd = [None] * _G_NCH
        gd[0] = start_gather(0)
        for c in range(_G_NCH):
            if c + 1 < _G_NCH:
                if c + 1 >= 2:
                    wd[c - 1].wait()
                gd[c + 1] = start_gather(c + 1)
            gd[c].wait()
            wd[c] = pltpu.async_copy(
                rows[c % 2],
                xg_hbm.at[pl.ds(rbase + _G_CHUNK * c, _G_CHUNK)],
                wsem[c % 2])
        wd[_G_NCH - 2].wait()
        wd[_G_NCH - 1].wait()

    return _gather_body


def _gather(nhalf, sorted_tok, x_flat):
    return pl.kernel(
        _make_gather_body(nhalf),
        out_type=jax.ShapeDtypeStruct((PH, D), jnp.float32),
        mesh=plsc.VectorSubcoreMesh(core_axis_name="c", subcore_axis_name="s"),
        scratch_types=[
            pltpu.VMEM((_G_ROWS,), jnp.int32),
            pltpu.VMEM((_G_CHUNK, D), jnp.float32),
            pltpu.VMEM((_G_CHUNK, D), jnp.float32),
            pltpu.SemaphoreType.DMA,
            pltpu.SemaphoreType.DMA,
            pltpu.SemaphoreType.DMA,
            pltpu.SemaphoreType.DMA,
        ],
        name=f"gather_half{nhalf}",
    )(sorted_tok, x_flat)


def _ffn1_body(be_ref, xg_ref, w1_ref, w3_ref, h_ref):
    xb = xg_ref[...]
    g = jnp.dot(xb, w1_ref[0], preferred_element_type=jnp.float32)
    u = jnp.dot(xb, w3_ref[0], preferred_element_type=jnp.float32)
    h_ref[...] = (g * jax.lax.logistic(g)) * u


def _ffn1(nhalf, beh, xg_h, w1, w3):
    boff = nhalf * NBH
    return pl.pallas_call(
        _ffn1_body,
        grid_spec=pltpu.PrefetchScalarGridSpec(
            num_scalar_prefetch=1,
            grid=(DF // BDF, NBH),
            in_specs=[
                pl.BlockSpec((BM, D), lambda j, b, be: (b, 0)),
                pl.BlockSpec((1, D, BDF),
                             lambda j, b, be: (be[b + boff], 0, j)),
                pl.BlockSpec((1, D, BDF),
                             lambda j, b, be: (be[b + boff], 0, j)),
            ],
            out_specs=pl.BlockSpec((BM, BDF), lambda j, b, be: (b, j)),
        ),
        out_shape=jax.ShapeDtypeStruct((PH, DF), jnp.float32),
        compiler_params=pltpu.CompilerParams(
            dimension_semantics=("arbitrary", "arbitrary")),
        name=f"ffn1_half{nhalf}",
    )(beh, xg_h, w1, w3)


def _ffn2_body(be_ref, h0_ref, h1_ref, w2_ref, s_ref, ys_ref):
    b = pl.program_id(0)

    @pl.when(b < NBH)
    def _():
        out = jnp.dot(h0_ref[...], w2_ref[0],
                      preferred_element_type=jnp.float32)
        ys_ref[...] = out * s_ref[...]

    @pl.when(b >= NBH)
    def _():
        out = jnp.dot(h1_ref[...], w2_ref[0],
                      preferred_element_type=jnp.float32)
        ys_ref[...] = out * s_ref[...]


def _ffn2(beh, h0, h1, w2, ssc):
    return pl.pallas_call(
        _ffn2_body,
        grid_spec=pltpu.PrefetchScalarGridSpec(
            num_scalar_prefetch=1,
            grid=(NB,),
            in_specs=[
                pl.BlockSpec((BM, DF),
                             lambda b, be: (jnp.minimum(b, NBH - 1), 0)),
                pl.BlockSpec((BM, DF),
                             lambda b, be: (jnp.maximum(b - NBH, 0), 0)),
                pl.BlockSpec((1, DF, D), lambda b, be: (be[b], 0, 0)),
                pl.BlockSpec((BM, 1), lambda b, be: (b, 0)),
            ],
            out_specs=pl.BlockSpec((BM, D), lambda b, be: (b, 0)),
        ),
        out_shape=jax.ShapeDtypeStruct((PMAX, D), jnp.float32),
        compiler_params=pltpu.CompilerParams(
            dimension_semantics=("arbitrary",)),
    )(beh, h0, h1, w2, ssc)


_C_TOK = N // 32
_C_CHUNK = 16
_C_NCH = _C_TOK // _C_CHUNK


def _combine_body(inv_hbm, ys_hbm, out_hbm, idxall_v, rows0, rows1,
                  out0, out1, g0, g1, w0, w1):
    wid = lax.axis_index("s") * 2 + lax.axis_index("c")
    tokbase = wid * _C_TOK
    rows = (rows0, rows1)
    outs = (out0, out1)
    gsem = (g0, g1)
    wsem = (w0, w1)
    pltpu.sync_copy(inv_hbm.at[pl.ds(2 * tokbase, 2 * _C_TOK)], idxall_v)

    def start_gather(c):
        return pltpu.async_copy(
            ys_hbm.at[idxall_v.at[pl.ds(2 * _C_CHUNK * c, 2 * _C_CHUNK)]],
            rows[c % 2], gsem[c % 2])

    gd = [None] * _C_NCH
    wd = [None] * _C_NCH
    gd[0] = start_gather(0)
    for c in range(_C_NCH):
        if c + 1 < _C_NCH:
            gd[c + 1] = start_gather(c + 1)
        gd[c].wait()
        if c >= 2:
            wd[c - 2].wait()
        rv, ov = rows[c % 2], outs[c % 2]
        for t in range(_C_CHUNK):
            def qbody(q, _, t=t):
                off = q * 16
                a = rv[2 * t, pl.ds(off, 16)]
                bb = rv[2 * t + 1, pl.ds(off, 16)]
                ov[t, pl.ds(off, 16)] = a + bb
                return 0
            lax.fori_loop(0, D // 16, qbody, 0)
        wd[c] = pltpu.async_copy(
            ov, out_hbm.at[pl.ds(tokbase + _C_CHUNK * c, _C_CHUNK)],
            wsem[c % 2])
    wd[_C_NCH - 2].wait()
    wd[_C_NCH - 1].wait()


def _combine(inv_f, ys):
    return pl.kernel(
        _combine_body,
        out_type=jax.ShapeDtypeStruct((N, D), jnp.float32),
        mesh=plsc.VectorSubcoreMesh(core_axis_name="c", subcore_axis_name="s"),
        scratch_types=[
            pltpu.VMEM((2 * _C_TOK,), jnp.int32),
            pltpu.VMEM((2 * _C_CHUNK, D), jnp.float32),
            pltpu.VMEM((2 * _C_CHUNK, D), jnp.float32),
            pltpu.VMEM((_C_CHUNK, D), jnp.float32),
            pltpu.VMEM((_C_CHUNK, D), jnp.float32),
            pltpu.SemaphoreType.DMA,
            pltpu.SemaphoreType.DMA,
            pltpu.SemaphoreType.DMA,
            pltpu.SemaphoreType.DMA,
        ],
    )(inv_f, ys)


def kernel(x, W_router, w1, w3, w2, expert_bias):
    Bs, Ss, Ds = x.shape
    x_flat = x.reshape(N, D)
    bias2 = expert_bias.reshape(1, E)

    eid, rank, scl, cnt = _router(x_flat, W_router, bias2)
    sorted_tok, sorted_scl, inv3, beh = _sort(
        eid.reshape(-1), rank.reshape(-1), scl.reshape(-1), cnt.reshape(16))
    xg0 = _gather(0, sorted_tok, x_flat)
    h0 = _ffn1(0, beh, xg0, w1, w3)
    xg1 = _gather(1, sorted_tok, x_flat)
    h1 = _ffn1(1, beh, xg1, w1, w3)
    ys = _ffn2(beh, h0, h1, w2, sorted_scl.reshape(PMAX, 1))
    out_flat = _combine(inv3.reshape(-1), ys)

    out = out_flat.reshape(Bs, Ss, Ds)
    aux_loss = jnp.zeros((), dtype=x.dtype)
    return (out, aux_loss)


# speedup vs baseline: 1.4569x; 1.0427x over previous
import jax
import jax.numpy as jnp
from jax import lax
from jax.experimental import pallas as pl
from jax.experimental.pallas import tpu as pltpu
from jax.experimental.pallas import tpu_sc as plsc

N = 4096
D = 1024
DF = 4096
E = 8
NA = 2 * N
BM = 256
PMAX = NA + E * BM
NB = PMAX // BM
NBE = 48
TB = 512
BDF = 2048
NEG = -1e30


def _router_body(x_ref, wr_ref, bias_ref, eid_ref, rank_ref, scl_ref,
                 cnt_ref, carry_ref):
    b = pl.program_id(0)

    @pl.when(b == 0)
    def _():
        carry_ref[...] = jnp.zeros_like(carry_ref)

    xb = x_ref[...]
    logits = jax.lax.dot_general(
        xb, wr_ref[...], (((1,), (1,)), ((), ())),
        preferred_element_type=jnp.float32)
    logits = logits + bias_ref[...]
    iota_e = lax.broadcasted_iota(jnp.int32, (TB, E), 1)
    m1 = jnp.max(logits, axis=1, keepdims=True)
    a1 = jnp.min(jnp.where(logits == m1, iota_e, E), axis=1, keepdims=True)
    masked = jnp.where(iota_e == a1, NEG, logits)
    m2 = jnp.max(masked, axis=1, keepdims=True)
    a2 = jnp.min(jnp.where(masked == m2, iota_e, E), axis=1, keepdims=True)
    s0 = 1.0 / (1.0 + jnp.exp(m2 - m1))
    s1 = 1.0 - s0

    oh1 = (iota_e == a1).astype(jnp.float32)
    oh2 = (iota_e == a2).astype(jnp.float32)
    oh = oh1 + oh2
    r_i = lax.broadcasted_iota(jnp.int32, (TB, TB), 0)
    c_i = lax.broadcasted_iota(jnp.int32, (TB, TB), 1)
    stril = (r_i > c_i).astype(jnp.float32)
    carry = carry_ref[0:1, 0:E]
    rank_base = jax.lax.dot_general(
        stril, oh, (((1,), (0,)), ((), ())),
        preferred_element_type=jnp.float32) + carry
    rank0 = jnp.sum(rank_base * oh1, axis=1, keepdims=True)
    rank1 = jnp.sum(rank_base * oh2, axis=1, keepdims=True)

    eid_ref[...] = jnp.concatenate([a1, a2], axis=1)
    rank_ref[...] = jnp.concatenate([rank0, rank1], axis=1).astype(jnp.int32)
    scl_ref[...] = jnp.concatenate([s0, s1], axis=1)
    new_c = carry + jnp.sum(oh, axis=0, keepdims=True)
    carry_ref[0:1, 0:E] = new_c
    cnt = jnp.concatenate(
        [new_c, jnp.zeros((1, 16 - E), jnp.float32)], axis=1)
    cnt_ref[...] = cnt.astype(jnp.int32)


def _router(x_flat, W_router, bias2):
    return pl.pallas_call(
        _router_body,
        grid=(N // TB,),
        in_specs=[
            pl.BlockSpec((TB, D), lambda b: (b, 0)),
            pl.BlockSpec((E, D), lambda b: (0, 0)),
            pl.BlockSpec((1, E), lambda b: (0, 0)),
        ],
        out_specs=[
            pl.BlockSpec((TB, 2), lambda b: (b, 0)),
            pl.BlockSpec((TB, 2), lambda b: (b, 0)),
            pl.BlockSpec((TB, 2), lambda b: (b, 0)),
            pl.BlockSpec((1, 16), lambda b: (0, 0)),
        ],
        out_shape=[
            jax.ShapeDtypeStruct((N, 2), jnp.int32),
            jax.ShapeDtypeStruct((N, 2), jnp.int32),
            jax.ShapeDtypeStruct((N, 2), jnp.float32),
            jax.ShapeDtypeStruct((1, 16), jnp.int32),
        ],
        scratch_shapes=[pltpu.VMEM((8, 128), jnp.float32)],
        compiler_params=pltpu.CompilerParams(
            dimension_semantics=("arbitrary",)),
    )(x_flat, W_router, bias2)


_AS_PER_TILE = NA // 16
_SL_PER_TILE = PMAX // 16


def _sort_body(eid_hbm, rank_hbm, scl_hbm, cnt_hbm,
               st_hbm, ss_hbm, inv_hbm, beh_hbm,
               cnt_v, padoff_v, blkoff_v, eid_v, rank_v, scl_v,
               pos_v, tok_v, sc4_v, tmp_v, tmpf_v, be_v, sh_t, sh_s):
    cid = lax.axis_index("c")
    sid = lax.axis_index("s")

    @pl.when(cid == 0)
    def _():
        base = sid * _AS_PER_TILE
        pltpu.sync_copy(cnt_hbm, cnt_v)
        c = cnt_v[...]
        pc = lax.shift_left(lax.shift_right_logical(c + (BM - 1), 8), 8)
        incl = jnp.cumsum(pc)
        padoff = incl - pc
        padoff_v[...] = padoff
        blkoff_v[...] = lax.shift_right_logical(padoff, 8)
        pltpu.sync_copy(eid_hbm.at[pl.ds(base, _AS_PER_TILE)], eid_v)
        pltpu.sync_copy(rank_hbm.at[pl.ds(base, _AS_PER_TILE)], rank_v)
        pltpu.sync_copy(scl_hbm.at[pl.ds(base, _AS_PER_TILE)], scl_v)

        iota16 = lax.iota(jnp.int32, 16)
        for m in range(_AS_PER_TILE // 16):
            e = eid_v[pl.ds(16 * m, 16)]
            r = rank_v[pl.ds(16 * m, 16)]
            off = plsc.load_gather(padoff_v, [e])
            pos = off + r
            tok = lax.shift_right_logical(base + 16 * m + iota16, 1)
            row, col = m // 8, (m % 8) * 16
            pos_v[row, pl.ds(col, 16)] = pos
            tok_v[row, pl.ds(col, 16)] = tok
            sc4_v[row, pl.ds(col, 16)] = scl_v[pl.ds(16 * m, 16)]

        pltpu.sync_copy(pos_v, inv_hbm.at[sid])

        z16 = jnp.zeros((16,), jnp.int32)
        zf16 = jnp.zeros((16,), jnp.float32)
        for m in range(_SL_PER_TILE // 16):
            tmp_v[pl.ds(16 * m, 16)] = z16
            tmpf_v[pl.ds(16 * m, 16)] = zf16
        sl = pl.ds(sid * _SL_PER_TILE, _SL_PER_TILE)
        pltpu.sync_copy(tmp_v, sh_t.at[sl])
        pltpu.sync_copy(tmpf_v, sh_s.at[sl])
        plsc.subcore_barrier()

        for j in range(4):
            pltpu.sync_copy(tok_v.at[j], sh_t.at[pos_v.at[j]], add=True)
            pltpu.sync_copy(sc4_v.at[j], sh_s.at[pos_v.at[j]], add=True)
        plsc.subcore_barrier()

        pltpu.sync_copy(sh_t.at[sl], tmp_v)
        pltpu.sync_copy(tmp_v, st_hbm.at[sl])
        pltpu.sync_copy(sh_s.at[sl], tmpf_v)
        pltpu.sync_copy(tmpf_v, ss_hbm.at[sl])

        @pl.when(sid == 0)
        def _():
            for m in range(NBE // 16):
                bv = lax.iota(jnp.int32, 16) + 16 * m
                acc = jnp.zeros((16,), jnp.int32)
                for e in range(E):
                    sp = plsc.load_gather(
                        blkoff_v, [jnp.full((16,), e, jnp.int32)])
                    acc = acc + jnp.where(bv >= sp, 1, 0).astype(jnp.int32)
                be_v[pl.ds(16 * m, 16)] = acc - 1
            pltpu.sync_copy(be_v, beh_hbm)


def _sort(eid_f, rank_f, scl_f, cnt16):
    return pl.kernel(
        _sort_body,
        out_type=[
            jax.ShapeDtypeStruct((PMAX,), jnp.int32),
            jax.ShapeDtypeStruct((PMAX,), jnp.float32),
            jax.ShapeDtypeStruct((16, 4, 128), jnp.int32),
            jax.ShapeDtypeStruct((NBE,), jnp.int32),
        ],
        mesh=plsc.VectorSubcoreMesh(core_axis_name="c", subcore_axis_name="s"),
        scratch_types=[
            pltpu.VMEM((16,), jnp.int32),
            pltpu.VMEM((16,), jnp.int32),
            pltpu.VMEM((16,), jnp.int32),
            pltpu.VMEM((_AS_PER_TILE,), jnp.int32),
            pltpu.VMEM((_AS_PER_TILE,), jnp.int32),
            pltpu.VMEM((_AS_PER_TILE,), jnp.float32),
            pltpu.VMEM((4, 128), jnp.int32),
            pltpu.VMEM((4, 128), jnp.int32),
            pltpu.VMEM((4, 128), jnp.float32),
            pltpu.VMEM((_SL_PER_TILE,), jnp.int32),
            pltpu.VMEM((_SL_PER_TILE,), jnp.float32),
            pltpu.VMEM((NBE,), jnp.int32),
            pltpu.VMEM_SHARED((PMAX,), jnp.int32),
            pltpu.VMEM_SHARED((PMAX,), jnp.float32),
        ],
        compiler_params=pltpu.CompilerParams(needs_layout_passes=False),
    )(eid_f, rank_f, scl_f, cnt16)


PH = PMAX // 2
NBH = NB // 2
_G_ROWS = PH // 32
_G_CHUNK = 40
_G_NCH = _G_ROWS // _G_CHUNK


def _make_gather_body(nhalf):
    def _gather_body(st_hbm, x_hbm, xg_hbm, idxall_v, rows0, rows1,
                     g0, g1, w0, w1):
        wid = lax.axis_index("s") * 2 + lax.axis_index("c")
        sbase = nhalf * PH + wid * _G_ROWS
        rbase = wid * _G_ROWS
        rows = (rows0, rows1)
        gsem = (g0, g1)
        wsem = (w0, w1)
        pltpu.sync_copy(st_hbm.at[pl.ds(sbase, _G_ROWS)], idxall_v)

        def start_gather(c):
            return pltpu.async_copy(
                x_hbm.at[idxall_v.at[pl.ds(_G_CHUNK * c, _G_CHUNK)]],
                rows[c % 2], gsem[c % 2])

        gd = [None] * _G_NCH
        wd = [None] * _G_NCH
        gd[0] = start_gather(0)
        for c in range(_G_NCH):
            if c + 1 < _G_NCH:
                if c + 1 >= 2:
                    wd[c - 1].wait()
                gd[c + 1] = start_gather(c + 1)
            gd[c].wait()
            wd[c] = pltpu.async_copy(
                rows[c % 2],
                xg_hbm.at[pl.ds(rbase + _G_CHUNK * c, _G_CHUNK)],
                wsem[c % 2])
        wd[_G_NCH - 2].wait()
        wd[_G_NCH - 1].wait()

    return _gather_body


def _gather(nhalf, sorted_tok, x_flat):
    return pl.kernel(
        _make_gather_body(nhalf),
        out_type=jax.ShapeDtypeStruct((PH, D), jnp.float32),
        mesh=plsc.VectorSubcoreMesh(core_axis_name="c", subcore_axis_name="s"),
        scratch_types=[
            pltpu.VMEM((_G_ROWS,), jnp.int32),
            pltpu.VMEM((_G_CHUNK, D), jnp.float32),
            pltpu.VMEM((_G_CHUNK, D), jnp.float32),
            pltpu.SemaphoreType.DMA,
            pltpu.SemaphoreType.DMA,
            pltpu.SemaphoreType.DMA,
            pltpu.SemaphoreType.DMA,
        ],
        name=f"gather_half{nhalf}",
    )(sorted_tok, x_flat)


def _ffn1_body(be_ref, xg_ref, w1_ref, w3_ref, h_ref):
    xb = xg_ref[...]
    g = jnp.dot(xb, w1_ref[0], preferred_element_type=jnp.float32)
    u = jnp.dot(xb, w3_ref[0], preferred_element_type=jnp.float32)
    h_ref[...] = ((g * jax.lax.logistic(g)) * u).astype(jnp.bfloat16)


def _ffn1_body_alias(be_ref, xg_ref, w1_ref, w3_ref, hprev_ref, h_ref):
    _ffn1_body(be_ref, xg_ref, w1_ref, w3_ref, h_ref)


def _ffn1(nhalf, beh, xg_h, w1, w3, h_prev=None):
    boff = nhalf * NBH
    in_specs = [
        pl.BlockSpec((BM, D), lambda j, b, be: (b, 0)),
        pl.BlockSpec((1, D, BDF), lambda j, b, be: (be[b + boff], 0, j)),
        pl.BlockSpec((1, D, BDF), lambda j, b, be: (be[b + boff], 0, j)),
    ]
    args = [beh, xg_h, w1, w3]
    body = _ffn1_body
    aliases = {}
    if h_prev is not None:
        in_specs.append(pl.BlockSpec((BM, BDF), lambda j, b, be: (0, 0)))
        args.append(h_prev)
        body = _ffn1_body_alias
        aliases = {4: 0}
    return pl.pallas_call(
        body,
        grid_spec=pltpu.PrefetchScalarGridSpec(
            num_scalar_prefetch=1,
            grid=(DF // BDF, NBH),
            in_specs=in_specs,
            out_specs=pl.BlockSpec((BM, BDF),
                                   lambda j, b, be: (b + boff, j)),
        ),
        out_shape=jax.ShapeDtypeStruct((PMAX, DF), jnp.bfloat16),
        input_output_aliases=aliases,
        compiler_params=pltpu.CompilerParams(
            dimension_semantics=("arbitrary", "arbitrary")),
        name=f"ffn1_half{nhalf}",
    )(*args)


def _ffn2_body(be_ref, h_ref, w2_ref, s_ref, ys_ref):
    hb = h_ref[...].astype(jnp.float32)
    out = jnp.dot(hb, w2_ref[0], preferred_element_type=jnp.float32)
    ys_ref[...] = out * s_ref[...]


def _ffn2(beh, h, w2, ssc):
    return pl.pallas_call(
        _ffn2_body,
        grid_spec=pltpu.PrefetchScalarGridSpec(
            num_scalar_prefetch=1,
            grid=(NB,),
            in_specs=[
                pl.BlockSpec((BM, DF), lambda b, be: (b, 0)),
                pl.BlockSpec((1, DF, D), lambda b, be: (be[b], 0, 0)),
                pl.BlockSpec((BM, 1), lambda b, be: (b, 0)),
            ],
            out_specs=pl.BlockSpec((BM, D), lambda b, be: (b, 0)),
        ),
        out_shape=jax.ShapeDtypeStruct((PMAX, D), jnp.float32),
        compiler_params=pltpu.CompilerParams(
            dimension_semantics=("arbitrary",)),
    )(beh, h, w2, ssc)


_C_TOK = N // 32
_C_CHUNK = 16
_C_NCH = _C_TOK // _C_CHUNK


def _combine_body(inv_hbm, ys_hbm, out_hbm, idxall_v, rows0, rows1,
                  out0, out1, g0, g1, w0, w1):
    wid = lax.axis_index("s") * 2 + lax.axis_index("c")
    tokbase = wid * _C_TOK
    rows = (rows0, rows1)
    outs = (out0, out1)
    gsem = (g0, g1)
    wsem = (w0, w1)
    pltpu.sync_copy(inv_hbm.at[pl.ds(2 * tokbase, 2 * _C_TOK)], idxall_v)

    def start_gather(c):
        return pltpu.async_copy(
            ys_hbm.at[idxall_v.at[pl.ds(2 * _C_CHUNK * c, 2 * _C_CHUNK)]],
            rows[c % 2], gsem[c % 2])

    gd = [None] * _C_NCH
    wd = [None] * _C_NCH
    gd[0] = start_gather(0)
    for c in range(_C_NCH):
        if c + 1 < _C_NCH:
            gd[c + 1] = start_gather(c + 1)
        gd[c].wait()
        if c >= 2:
            wd[c - 2].wait()
        rv, ov = rows[c % 2], outs[c % 2]
        for t in range(_C_CHUNK):
            def qbody(q, _, t=t):
                off = q * 16
                a = rv[2 * t, pl.ds(off, 16)]
                bb = rv[2 * t + 1, pl.ds(off, 16)]
                ov[t, pl.ds(off, 16)] = a + bb
                return 0
            lax.fori_loop(0, D // 16, qbody, 0)
        wd[c] = pltpu.async_copy(
            ov, out_hbm.at[pl.ds(tokbase + _C_CHUNK * c, _C_CHUNK)],
            wsem[c % 2])
    wd[_C_NCH - 2].wait()
    wd[_C_NCH - 1].wait()


def _combine(inv_f, ys):
    return pl.kernel(
        _combine_body,
        out_type=jax.ShapeDtypeStruct((N, D), jnp.float32),
        mesh=plsc.VectorSubcoreMesh(core_axis_name="c", subcore_axis_name="s"),
        scratch_types=[
            pltpu.VMEM((2 * _C_TOK,), jnp.int32),
            pltpu.VMEM((2 * _C_CHUNK, D), jnp.float32),
            pltpu.VMEM((2 * _C_CHUNK, D), jnp.float32),
            pltpu.VMEM((_C_CHUNK, D), jnp.float32),
            pltpu.VMEM((_C_CHUNK, D), jnp.float32),
            pltpu.SemaphoreType.DMA,
            pltpu.SemaphoreType.DMA,
            pltpu.SemaphoreType.DMA,
            pltpu.SemaphoreType.DMA,
        ],
    )(inv_f, ys)


def kernel(x, W_router, w1, w3, w2, expert_bias):
    Bs, Ss, Ds = x.shape
    x_flat = x.reshape(N, D)
    bias2 = expert_bias.reshape(1, E)

    eid, rank, scl, cnt = _router(x_flat, W_router, bias2)
    sorted_tok, sorted_scl, inv3, beh = _sort(
        eid.reshape(-1), rank.reshape(-1), scl.reshape(-1), cnt.reshape(16))
    xg0 = _gather(0, sorted_tok, x_flat)
    h0 = _ffn1(0, beh, xg0, w1, w3)
    xg1 = _gather(1, sorted_tok, x_flat)
    h = _ffn1(1, beh, xg1, w1, w3, h_prev=h0)
    ys = _ffn2(beh, h, w2, sorted_scl.reshape(PMAX, 1))
    out_flat = _combine(inv3.reshape(-1), ys)

    out = out_flat.reshape(Bs, Ss, Ds)
    aux_loss = jnp.zeros((), dtype=x.dtype)
    return (out, aux_loss)
